# baseline (speedup 1.0000x reference)
import functools

import jax
import jax.numpy as jnp
from jax import lax
from jax.experimental import pallas as pl
from jax.experimental.pallas import tpu as pltpu
from jax.experimental.pallas import tpu_sc as plsc

MAXLEN = 8192
EMBED_DIM = 768
BATCH = 4
NROWS = BATCH * MAXLEN

T_SC = MAXLEN
ROWS_TC = NROWS - T_SC

NUM_CORES = 2
NUM_SUBCORES = 16
NW = NUM_CORES * NUM_SUBCORES
ROWS_PER_W = T_SC // NW
SB = 32
NU = ROWS_PER_W // SB
UNROLL = 8

_mesh = plsc.VectorSubcoreMesh(core_axis_name="c", subcore_axis_name="s")


@functools.partial(
    pl.kernel,
    mesh=_mesh,
    out_type=jax.ShapeDtypeStruct((NROWS, EMBED_DIM), jnp.float32),
    scratch_types=[
        pltpu.VMEM((SB, EMBED_DIM), jnp.float32),
        pltpu.VMEM((SB, EMBED_DIM), jnp.float32),
        pltpu.VMEM((SB, EMBED_DIM), jnp.float32),
        pltpu.VMEM((SB, EMBED_DIM), jnp.float32),
        pltpu.SemaphoreType.DMA,
        pltpu.SemaphoreType.DMA,
        pltpu.SemaphoreType.DMA,
        pltpu.SemaphoreType.DMA,
        pltpu.SemaphoreType.DMA,
        pltpu.SemaphoreType.DMA,
    ],
)
def _sc_body(x_hbm, pos_hbm, out_hbm,
             xb0, xb1, pb0, pb1, si0, si1, sp0, sp1, so0, so1):
    wid = lax.axis_index("s") * NUM_CORES + lax.axis_index("c")
    base = wid * ROWS_PER_W
    xbufs, pbufs = [xb0, xb1], [pb0, pb1]
    sin, spos, sout = [si0, si1], [sp0, sp1], [so0, so1]

    def start_in(u):
        return pltpu.async_copy(
            x_hbm.at[pl.ds(ROWS_TC + base + u * SB, SB)], xbufs[u % 2],
            sin[u % 2])

    def start_pos(u):
        return pltpu.async_copy(
            pos_hbm.at[pl.ds(base + u * SB, SB)], pbufs[u % 2], spos[u % 2])

    def start_out(u):
        return pltpu.async_copy(
            xbufs[u % 2], out_hbm.at[pl.ds(ROWS_TC + base + u * SB, SB)],
            sout[u % 2])

    def compute(u):
        xb, pb = xbufs[u % 2], pbufs[u % 2]

        def row_body(r, carry):
            @plsc.parallel_loop(0, EMBED_DIM, step=16, unroll=UNROLL)
            def _(o):
                plsc.addupdate(xb.at[r, pl.ds(o, 16)], pb[r, pl.ds(o, 16)])

            return carry

        lax.fori_loop(0, SB, row_body, 0)

    in_d, pos_d, out_d = {}, {}, {}
    in_d[0] = start_in(0)
    pos_d[0] = start_pos(0)
    for u in range(NU):
        if u + 1 < NU:
            if u >= 1:
                out_d.pop(u - 1).wait()
            in_d[u + 1] = start_in(u + 1)
            pos_d[u + 1] = start_pos(u + 1)
        in_d.pop(u).wait()
        pos_d.pop(u).wait()
        compute(u)
        out_d[u] = start_out(u)
    for r in sorted(out_d):
        out_d.pop(r).wait()


TC_BLK = 512
PER_SEQ = MAXLEN // TC_BLK


def _tc_body(x_ref, pos_ref, acc_ref, out_ref):
    out_ref[...] = x_ref[...] + pos_ref[...]


def _tc_part(x2, pos_table, acc):
    nblk = ROWS_TC // TC_BLK
    return pl.pallas_call(
        _tc_body,
        grid=(nblk,),
        in_specs=[
            pl.BlockSpec((TC_BLK, EMBED_DIM), lambda i: (i, 0)),
            pl.BlockSpec((TC_BLK, EMBED_DIM), lambda i: (i % PER_SEQ, 0)),
            pl.BlockSpec((8, 128), lambda i: (0, 0)),
        ],
        out_specs=pl.BlockSpec((TC_BLK, EMBED_DIM), lambda i: (i, 0)),
        out_shape=jax.ShapeDtypeStruct((NROWS, EMBED_DIM), jnp.float32),
        input_output_aliases={2: 0},
    )(x2, pos_table, acc)


def kernel(x, pos_table):
    x2 = x.reshape(NROWS, EMBED_DIM)
    acc = _sc_body(x2, pos_table)
    out = _tc_part(x2, pos_table, acc)
    return out.reshape(BATCH, MAXLEN, EMBED_DIM)


# speedup vs baseline: 1.0112x; 1.0112x over previous
import functools

import jax
import jax.numpy as jnp
from jax import lax
from jax.experimental import pallas as pl
from jax.experimental.pallas import tpu as pltpu
from jax.experimental.pallas import tpu_sc as plsc

MAXLEN = 8192
EMBED_DIM = 768
BATCH = 4
NROWS = BATCH * MAXLEN

NUM_CORES = 2
NUM_SUBCORES = 16
NW = NUM_CORES * NUM_SUBCORES
SEQ_PER_W = MAXLEN // NW
SB = 32
NBLK = SEQ_PER_W // SB
NU = NBLK * BATCH
UNROLL = 8

_mesh = plsc.VectorSubcoreMesh(core_axis_name="c", subcore_axis_name="s")


@functools.partial(
    pl.kernel,
    mesh=_mesh,
    out_type=jax.ShapeDtypeStruct((NROWS, EMBED_DIM), jnp.float32),
    scratch_types=[
        pltpu.VMEM((SB, EMBED_DIM), jnp.float32),
        pltpu.VMEM((SB, EMBED_DIM), jnp.float32),
        pltpu.VMEM((SB, EMBED_DIM), jnp.float32),
        pltpu.VMEM((SB, EMBED_DIM), jnp.float32),
        pltpu.VMEM((SB, EMBED_DIM), jnp.float32),
        pltpu.SemaphoreType.DMA,
        pltpu.SemaphoreType.DMA,
        pltpu.SemaphoreType.DMA,
        pltpu.SemaphoreType.DMA,
        pltpu.SemaphoreType.DMA,
        pltpu.SemaphoreType.DMA,
        pltpu.SemaphoreType.DMA,
        pltpu.SemaphoreType.DMA,
    ],
)
def _sc_body(x_hbm, pos_hbm, out_hbm,
             xb0, xb1, xb2, pb0, pb1,
             si0, si1, si2, sp0, sp1, so0, so1, so2):
    wid = lax.axis_index("s") * NUM_CORES + lax.axis_index("c")
    base = wid * SEQ_PER_W
    xbufs, pbufs = [xb0, xb1, xb2], [pb0, pb1]
    sin, spos, sout = [si0, si1, si2], [sp0, sp1], [so0, so1, so2]

    def x_row0(u):
        blk, b = divmod(u, BATCH)
        return b * MAXLEN + base + blk * SB

    def start_in(u):
        return pltpu.async_copy(
            x_hbm.at[pl.ds(x_row0(u), SB)], xbufs[u % 3], sin[u % 3])

    def start_pos(blk):
        return pltpu.async_copy(
            pos_hbm.at[pl.ds(base + blk * SB, SB)], pbufs[blk % 2],
            spos[blk % 2])

    def start_out(u):
        return pltpu.async_copy(
            xbufs[u % 3], out_hbm.at[pl.ds(x_row0(u), SB)], sout[u % 3])

    def compute(u, blk):
        xb, pb = xbufs[u % 3], pbufs[blk % 2]

        def row_body(r, carry):
            @plsc.parallel_loop(0, EMBED_DIM, step=16, unroll=UNROLL)
            def _(o):
                plsc.addupdate(xb.at[r, pl.ds(o, 16)], pb[r, pl.ds(o, 16)])

            return carry

        lax.fori_loop(0, SB, row_body, 0)

    in_d, pos_d, out_d = {}, {}, {}
    pos_d[0] = start_pos(0)
    pos_d[1] = start_pos(1)
    in_d[0] = start_in(0)
    in_d[1] = start_in(1)
    for u in range(NU):
        blk, b = divmod(u, BATCH)
        if u + 2 < NU:
            if u >= 1:
                out_d.pop(u - 1).wait()
            in_d[u + 2] = start_in(u + 2)
        in_d.pop(u).wait()
        if b == 0:
            pos_d.pop(blk).wait()
        compute(u, blk)
        if b == BATCH - 1 and blk + 2 < NBLK:
            pos_d[blk + 2] = start_pos(blk + 2)
        out_d[u] = start_out(u)
    for r in sorted(out_d):
        out_d.pop(r).wait()


def kernel(x, pos_table):
    x2 = x.reshape(NROWS, EMBED_DIM)
    out = _sc_body(x2, pos_table)
    return out.reshape(BATCH, MAXLEN, EMBED_DIM)
